# baseline (speedup 1.0000x reference)
import jax
import jax.numpy as jnp
from jax import lax
from jax.experimental import pallas as pl
from jax.experimental.pallas import tpu as pltpu
from jax.experimental.pallas import tpu_sc as plsc

N = 4096
B = 64
D = 4
L = 16
NCH = N // L
SCH = 128
NSC = N // SCH
HLEN = N + L

_INV_N = 1.0 / float(N)


def _resample_body(w_hbm, u_hbm, state_hbm, out_hbm,
                   w_v, u_v, x_v, hist_v, o_v,
                   sw0, sw1, sx0, sx1, su, so0, so1):
    nc = 2
    wid = lax.axis_index("s") * nc + lax.axis_index("c")

    zf = jnp.zeros((L,), jnp.float32)
    zi = jnp.zeros((L,), jnp.int32)
    ones_i = jnp.ones((L,), jnp.int32)
    i128 = lax.iota(jnp.int32, L) * SCH

    b0 = wid * 2
    hu = pltpu.async_copy(u_hbm, u_v, su)
    hw = [pltpu.async_copy(w_hbm.at[pl.ds((b0 + i) * N, N)],
                           w_v.at[pl.ds(i * N, N)], s)
          for i, s in ((0, sw0), (1, sw1))]
    hx = [pltpu.async_copy(state_hbm.at[pl.ds((b0 + i) * D * N, D * N)],
                           x_v.at[pl.ds(i * D * N, D * N)], s)
          for i, s in ((0, sx0), (1, sx1))]
    ho = [None, None]
    osem = (so0, so1)

    def zero_body(c, _):
        base = pl.multiple_of(c * 4 * L, L)
        for j in range(4):
            hist_v[pl.ds(base + j * L, L)] = zi
        return 0

    lax.fori_loop(0, (HLEN // L) // 4, zero_body, 0)
    hist_v[pl.ds(N, L)] = zi
    hu.wait()

    for bi in range(2):
        b = b0 + bi
        woff = bi * N
        xoff = bi * D * N
        hw[bi].wait()
        u = plsc.load_gather(u_v, [jnp.full((L,), b, jnp.int32)])

        def scan_body(i, carry):
            a0, a1 = carry
            ib = i * 4
            for di in range(4):
                idx0 = i128 + (woff + ib + di)
                idx1 = i128 + (woff + L * SCH + ib + di)
                a0 = a0 + plsc.load_gather(w_v, [idx0])
                a1 = a1 + plsc.load_gather(w_v, [idx1])
                plsc.store_scatter(w_v, [idx0], a0)
                plsc.store_scatter(w_v, [idx1], a1)
            return (a0, a1)

        lax.fori_loop(0, SCH // 4, scan_body, (zf, zf))

        def hist_body(c, off):
            mbase = c * SCH
            for j in range(SCH // L):
                base = pl.multiple_of(woff + mbase + j * L, L)
                cdf = w_v[pl.ds(base, L)] + off
                mf4 = cdf * float(N)
                mf = mf4 - u
                it = mf.astype(jnp.int32)
                ifl = it - (it.astype(jnp.float32) > mf).astype(jnp.int32)
                c0 = ifl + 1
                c1 = c0 - ((ifl.astype(jnp.float32) + u) > mf4).astype(jnp.int32)
                c2 = c1 + ((c1.astype(jnp.float32) + u) <= mf4).astype(jnp.int32)
                cc = jnp.minimum(c2, N + 8)
                plsc.addupdate_scatter(hist_v, [cc], ones_i)
            tot = plsc.load_gather(
                w_v, [jnp.full((L,), woff + mbase + (SCH - 1), jnp.int32)])
            return off + tot

        lax.fori_loop(0, NSC, hist_body, zf)

        def iscan_body(i, carry):
            a0, a1 = carry
            ib = i * 4
            for di in range(4):
                idx0 = i128 + (ib + di)
                idx1 = i128 + (L * SCH + ib + di)
                a0 = a0 + plsc.load_gather(hist_v, [idx0])
                a1 = a1 + plsc.load_gather(hist_v, [idx1])
                plsc.store_scatter(hist_v, [idx0], a0)
                plsc.store_scatter(hist_v, [idx1], a1)
            return (a0, a1)

        lax.fori_loop(0, SCH // 4, iscan_body, (zi, zi))

        hx[bi].wait()

        def res_body(c, ioff):
            mbase = c * SCH
            tot = plsc.load_gather(
                hist_v, [jnp.full((L,), mbase + (SCH - 1), jnp.int32)])
            for j in range(SCH // L):
                base = pl.multiple_of(mbase + j * L, L)
                ics = hist_v[pl.ds(base, L)] + ioff
                hist_v[pl.ds(base, L)] = zi
                idxv = jnp.minimum(ics, N - 1) + xoff
                for k in range(D):
                    g = plsc.load_gather(x_v, [idxv + k * N])
                    o_v[pl.ds(xoff + k * N + base, L)] = g
            return ioff + tot

        lax.fori_loop(0, NSC, res_body, zi)
        ho[bi] = pltpu.async_copy(o_v.at[pl.ds(xoff, D * N)],
                                  out_hbm.at[pl.ds(b * D * N, D * N)],
                                  osem[bi])
    ho[0].wait()
    ho[1].wait()


_mesh = plsc.VectorSubcoreMesh(core_axis_name="c", subcore_axis_name="s")

_resample = pl.kernel(
    _resample_body,
    out_type=jax.ShapeDtypeStruct((B * D * N,), jnp.float32),
    mesh=_mesh,
    compiler_params=pltpu.CompilerParams(needs_layout_passes=False,
                                         disable_bounds_checks=True),
    scratch_types=[
        pltpu.VMEM((2 * N,), jnp.float32),
        pltpu.VMEM((B,), jnp.float32),
        pltpu.VMEM((2 * D * N,), jnp.float32),
        pltpu.VMEM((HLEN,), jnp.int32),
        pltpu.VMEM((2 * D * N,), jnp.float32),
        pltpu.SemaphoreType.DMA,
        pltpu.SemaphoreType.DMA,
        pltpu.SemaphoreType.DMA,
        pltpu.SemaphoreType.DMA,
        pltpu.SemaphoreType.DMA,
        pltpu.SemaphoreType.DMA,
        pltpu.SemaphoreType.DMA,
    ],
)


def _obs_logw(y, x):
    return -0.5 * jnp.sum((y[None, :, :] - x) ** 2, axis=-1) / 0.25


def _norm(logw):
    lse = jax.scipy.special.logsumexp(logw, axis=0)
    return logw - lse[None, :]


def kernel(observation, init_noise, trans_noise, resample_u, A,
           n_particles, time_extent):
    state = init_noise
    logw = _obs_logw(observation[0], state)
    lw = _norm(logw)
    w = jnp.exp(lw)
    outputs = [jnp.sum(w[..., None] * state, axis=0)]

    for t in range(1, trans_noise.shape[0] + 1):
        w = jnp.exp(lw)
        w_t = w.T.reshape(B * N)
        state_t = state.transpose(1, 2, 0).reshape(B * D * N)
        res_t = _resample(w_t, resample_u[t - 1], state_t)
        resampled = res_t.reshape(B, D, N).transpose(2, 0, 1)
        state = jnp.einsum('nbd,kd->nbk', resampled, A) + 0.3 * trans_noise[t - 1]
        logw = _obs_logw(observation[t], state)
        lw = _norm(logw)
        w2 = jnp.exp(lw)
        outputs.append(jnp.sum(w2[..., None] * state, axis=0))

    return jnp.stack(outputs, axis=0)


# speedup vs baseline: 1.1310x; 1.1310x over previous
import jax
import jax.numpy as jnp
from jax import lax
from jax.experimental import pallas as pl
from jax.experimental.pallas import tpu as pltpu
from jax.experimental.pallas import tpu_sc as plsc

N = 4096
B = 64
D = 4
L = 16
NCH = N // L
SCH = 128
NSC = N // SCH
HLEN = N + L

_INV_N = 1.0 / float(N)


def _resample_body(w_hbm, u_hbm, state_hbm, out_hbm,
                   w_v, u_v, x_v, hist_v, o_v,
                   sw0, sw1, sx0, sx1, su, so0, so1):
    nc = 2
    wid = lax.axis_index("s") * nc + lax.axis_index("c")

    zf = jnp.zeros((L,), jnp.float32)
    zi = jnp.zeros((L,), jnp.int32)
    ones_i = jnp.ones((L,), jnp.int32)
    i128 = lax.iota(jnp.int32, L) * SCH

    b0 = wid * 2
    hu = pltpu.async_copy(u_hbm, u_v, su)
    hw = [pltpu.async_copy(w_hbm.at[pl.ds((b0 + i) * N, N)],
                           w_v.at[pl.ds(i * N, N)], s)
          for i, s in ((0, sw0), (1, sw1))]
    hx = [pltpu.async_copy(state_hbm.at[pl.ds((b0 + i) * D * N, D * N)],
                           x_v.at[pl.ds(i * D * N, D * N)], s)
          for i, s in ((0, sx0), (1, sx1))]
    ho = [None, None]
    osem = (so0, so1)

    def zero_body(c, _):
        base = pl.multiple_of(c * 4 * L, L)
        for j in range(4):
            hist_v[pl.ds(base + j * L, L)] = zi
        return 0

    lax.fori_loop(0, (HLEN // L) // 4, zero_body, 0)
    hist_v[pl.ds(N, L)] = zi
    hu.wait()

    for bi in range(2):
        b = b0 + bi
        woff = bi * N
        xoff = bi * D * N
        hw[bi].wait()
        u = plsc.load_gather(u_v, [jnp.full((L,), b, jnp.int32)])

        def scan_body(i, carry):
            a0, a1 = carry
            ib = i * 4
            for di in range(4):
                idx0 = i128 + (woff + ib + di)
                idx1 = i128 + (woff + L * SCH + ib + di)
                a0 = a0 + plsc.load_gather(w_v, [idx0])
                a1 = a1 + plsc.load_gather(w_v, [idx1])
                plsc.store_scatter(w_v, [idx0], a0)
                plsc.store_scatter(w_v, [idx1], a1)
            return (a0, a1)

        lax.fori_loop(0, SCH // 4, scan_body, (zf, zf))

        def hist_body(c, off):
            mbase = c * SCH
            for j in range(SCH // L):
                base = pl.multiple_of(woff + mbase + j * L, L)
                cdf = w_v[pl.ds(base, L)] + off
                mf4 = cdf * float(N)
                mf = mf4 - u
                it = mf.astype(jnp.int32)
                ifl = it - (it.astype(jnp.float32) > mf).astype(jnp.int32)
                c0 = ifl + 1
                c1 = c0 - ((ifl.astype(jnp.float32) + u) > mf4).astype(jnp.int32)
                c2 = c1 + ((c1.astype(jnp.float32) + u) <= mf4).astype(jnp.int32)
                cc = jnp.minimum(c2, N + 8)
                plsc.addupdate_scatter(hist_v, [cc], ones_i)
            tot = plsc.load_gather(
                w_v, [jnp.full((L,), woff + mbase + (SCH - 1), jnp.int32)])
            return off + tot

        lax.fori_loop(0, NSC, hist_body, zf)

        hx[bi].wait()

        def res_body(c, accc):
            base0 = c * 4 * L
            for j in range(4):
                base = pl.multiple_of(base0 + j * L, L)
                hv = hist_v[pl.ds(base, L)]
                hist_v[pl.ds(base, L)] = zi
                ics = plsc.cumsum(hv) + accc
                idxv = jnp.minimum(ics, N - 1) + xoff
                for k in range(D):
                    g = plsc.load_gather(x_v, [idxv + k * N])
                    o_v[pl.ds(xoff + k * N + base, L)] = g
                accc = ics[L - 1]
            return accc

        lax.fori_loop(0, NCH // 4, res_body, 0)
        ho[bi] = pltpu.async_copy(o_v.at[pl.ds(xoff, D * N)],
                                  out_hbm.at[pl.ds(b * D * N, D * N)],
                                  osem[bi])
    ho[0].wait()
    ho[1].wait()


_mesh = plsc.VectorSubcoreMesh(core_axis_name="c", subcore_axis_name="s")

_resample = pl.kernel(
    _resample_body,
    out_type=jax.ShapeDtypeStruct((B * D * N,), jnp.float32),
    mesh=_mesh,
    compiler_params=pltpu.CompilerParams(needs_layout_passes=False,
                                         disable_bounds_checks=True),
    scratch_types=[
        pltpu.VMEM((2 * N,), jnp.float32),
        pltpu.VMEM((B,), jnp.float32),
        pltpu.VMEM((2 * D * N,), jnp.float32),
        pltpu.VMEM((HLEN,), jnp.int32),
        pltpu.VMEM((2 * D * N,), jnp.float32),
        pltpu.SemaphoreType.DMA,
        pltpu.SemaphoreType.DMA,
        pltpu.SemaphoreType.DMA,
        pltpu.SemaphoreType.DMA,
        pltpu.SemaphoreType.DMA,
        pltpu.SemaphoreType.DMA,
        pltpu.SemaphoreType.DMA,
    ],
)


def _obs_logw(y, x):
    return -0.5 * jnp.sum((y[None, :, :] - x) ** 2, axis=-1) / 0.25


def _norm(logw):
    lse = jax.scipy.special.logsumexp(logw, axis=0)
    return logw - lse[None, :]


def kernel(observation, init_noise, trans_noise, resample_u, A,
           n_particles, time_extent):
    state = init_noise
    logw = _obs_logw(observation[0], state)
    lw = _norm(logw)
    w = jnp.exp(lw)
    outputs = [jnp.sum(w[..., None] * state, axis=0)]

    for t in range(1, trans_noise.shape[0] + 1):
        w = jnp.exp(lw)
        w_t = w.T.reshape(B * N)
        state_t = state.transpose(1, 2, 0).reshape(B * D * N)
        res_t = _resample(w_t, resample_u[t - 1], state_t)
        resampled = res_t.reshape(B, D, N).transpose(2, 0, 1)
        state = jnp.einsum('nbd,kd->nbk', resampled, A) + 0.3 * trans_noise[t - 1]
        logw = _obs_logw(observation[t], state)
        lw = _norm(logw)
        w2 = jnp.exp(lw)
        outputs.append(jnp.sum(w2[..., None] * state, axis=0))

    return jnp.stack(outputs, axis=0)


# speedup vs baseline: 1.1660x; 1.0309x over previous
import jax
import jax.numpy as jnp
from jax import lax
from jax.experimental import pallas as pl
from jax.experimental.pallas import tpu as pltpu
from jax.experimental.pallas import tpu_sc as plsc

N = 4096
B = 64
D = 4
L = 16
NCH = N // L
SCH = 128
NSC = N // SCH
HLEN = N + L

_INV_N = 1.0 / float(N)


def _resample_body(w_hbm, u_hbm, state_hbm, out_hbm,
                   w_v, u_v, x_v, hist_v, o_v,
                   sw0, sw1, sx0, sx1, su, so0, so1):
    nc = 2
    wid = lax.axis_index("s") * nc + lax.axis_index("c")

    zf = jnp.zeros((L,), jnp.float32)
    zi = jnp.zeros((L,), jnp.int32)
    ones_i = jnp.ones((L,), jnp.int32)
    i128 = lax.iota(jnp.int32, L) * SCH

    b0 = wid * 2
    hu = pltpu.async_copy(u_hbm, u_v, su)
    hw = [pltpu.async_copy(w_hbm.at[pl.ds((b0 + i) * N, N)],
                           w_v.at[pl.ds(i * N, N)], s)
          for i, s in ((0, sw0), (1, sw1))]
    hx = [pltpu.async_copy(state_hbm.at[pl.ds((b0 + i) * D * N, D * N)],
                           x_v.at[pl.ds(i * D * N, D * N)], s)
          for i, s in ((0, sx0), (1, sx1))]
    ho = [None, None]
    osem = (so0, so1)

    def zero_body(c, _):
        base = pl.multiple_of(c * 4 * L, L)
        for j in range(4):
            hist_v[pl.ds(base + j * L, L)] = zi
        return 0

    lax.fori_loop(0, (HLEN // L) // 4, zero_body, 0)
    hist_v[pl.ds(N, L)] = zi
    hu.wait()

    for bi in range(2):
        b = b0 + bi
        woff = bi * N
        xoff = bi * D * N
        hw[bi].wait()
        u = plsc.load_gather(u_v, [jnp.full((L,), b, jnp.int32)])

        def scan_body(i, carry):
            a0, a1 = carry
            ib = i * 8
            for di in range(8):
                idx0 = i128 + (woff + ib + di)
                idx1 = i128 + (woff + L * SCH + ib + di)
                a0 = a0 + plsc.load_gather(w_v, [idx0])
                a1 = a1 + plsc.load_gather(w_v, [idx1])
                plsc.store_scatter(w_v, [idx0], a0)
                plsc.store_scatter(w_v, [idx1], a1)
            return (a0, a1)

        lax.fori_loop(0, SCH // 8, scan_body, (zf, zf))

        def hist_body(c, off):
            mbase = c * SCH
            for j in range(SCH // L):
                base = pl.multiple_of(woff + mbase + j * L, L)
                cdf = w_v[pl.ds(base, L)] + off
                mf4 = cdf * float(N)
                mf = mf4 - u
                it = mf.astype(jnp.int32)
                ifl = it - (it.astype(jnp.float32) > mf).astype(jnp.int32)
                c0 = ifl + 1
                c1 = c0 - ((ifl.astype(jnp.float32) + u) > mf4).astype(jnp.int32)
                c2 = c1 + ((c1.astype(jnp.float32) + u) <= mf4).astype(jnp.int32)
                cc = jnp.minimum(c2, N + 8)
                plsc.addupdate_scatter(hist_v, [cc], ones_i)
            tot = plsc.load_gather(
                w_v, [jnp.full((L,), woff + mbase + (SCH - 1), jnp.int32)])
            return off + tot

        lax.fori_loop(0, NSC, hist_body, zf)

        hx[bi].wait()

        def res_body(c, accc):
            base0 = c * 8 * L
            for j in range(8):
                base = pl.multiple_of(base0 + j * L, L)
                hv = hist_v[pl.ds(base, L)]
                hist_v[pl.ds(base, L)] = zi
                ics = plsc.cumsum(hv) + accc
                idxv = jnp.minimum(ics, N - 1) + xoff
                for k in range(D):
                    g = plsc.load_gather(x_v, [idxv + k * N])
                    o_v[pl.ds(xoff + k * N + base, L)] = g
                accc = ics[L - 1]
            return accc

        lax.fori_loop(0, NCH // 8, res_body, 0)
        ho[bi] = pltpu.async_copy(o_v.at[pl.ds(xoff, D * N)],
                                  out_hbm.at[pl.ds(b * D * N, D * N)],
                                  osem[bi])
    ho[0].wait()
    ho[1].wait()


_mesh = plsc.VectorSubcoreMesh(core_axis_name="c", subcore_axis_name="s")

_resample = pl.kernel(
    _resample_body,
    out_type=jax.ShapeDtypeStruct((B * D * N,), jnp.float32),
    mesh=_mesh,
    compiler_params=pltpu.CompilerParams(needs_layout_passes=False,
                                         disable_bounds_checks=True),
    scratch_types=[
        pltpu.VMEM((2 * N,), jnp.float32),
        pltpu.VMEM((B,), jnp.float32),
        pltpu.VMEM((2 * D * N,), jnp.float32),
        pltpu.VMEM((HLEN,), jnp.int32),
        pltpu.VMEM((2 * D * N,), jnp.float32),
        pltpu.SemaphoreType.DMA,
        pltpu.SemaphoreType.DMA,
        pltpu.SemaphoreType.DMA,
        pltpu.SemaphoreType.DMA,
        pltpu.SemaphoreType.DMA,
        pltpu.SemaphoreType.DMA,
        pltpu.SemaphoreType.DMA,
    ],
)


def _obs_logw(y, x):
    return -0.5 * jnp.sum((y[None, :, :] - x) ** 2, axis=-1) / 0.25


def _norm(logw):
    lse = jax.scipy.special.logsumexp(logw, axis=0)
    return logw - lse[None, :]


def kernel(observation, init_noise, trans_noise, resample_u, A,
           n_particles, time_extent):
    state = init_noise
    logw = _obs_logw(observation[0], state)
    lw = _norm(logw)
    w = jnp.exp(lw)
    outputs = [jnp.sum(w[..., None] * state, axis=0)]

    for t in range(1, trans_noise.shape[0] + 1):
        w = jnp.exp(lw)
        w_t = w.T.reshape(B * N)
        state_t = state.transpose(1, 2, 0).reshape(B * D * N)
        res_t = _resample(w_t, resample_u[t - 1], state_t)
        resampled = res_t.reshape(B, D, N).transpose(2, 0, 1)
        state = jnp.einsum('nbd,kd->nbk', resampled, A) + 0.3 * trans_noise[t - 1]
        logw = _obs_logw(observation[t], state)
        lw = _norm(logw)
        w2 = jnp.exp(lw)
        outputs.append(jnp.sum(w2[..., None] * state, axis=0))

    return jnp.stack(outputs, axis=0)


# speedup vs baseline: 1.1736x; 1.0066x over previous
import jax
import jax.numpy as jnp
from jax import lax
from jax.experimental import pallas as pl
from jax.experimental.pallas import tpu as pltpu
from jax.experimental.pallas import tpu_sc as plsc

N = 4096
B = 64
D = 4
L = 16
NCH = N // L
SCH = 128
NSC = N // SCH
HLEN = N + L

_INV_N = 1.0 / float(N)


def _resample_body(w_hbm, u_hbm, state_hbm, out_hbm,
                   w_v, u_v, x_v, hist_v, o_v,
                   sw0, sw1, sx0, sx1, su, so0, so1):
    nc = 2
    wid = lax.axis_index("s") * nc + lax.axis_index("c")

    zf = jnp.zeros((L,), jnp.float32)
    zi = jnp.zeros((L,), jnp.int32)
    ones_i = jnp.ones((L,), jnp.int32)
    i128 = lax.iota(jnp.int32, L) * SCH

    b0 = wid * 2
    hu = pltpu.async_copy(u_hbm, u_v, su)
    hw = [pltpu.async_copy(w_hbm.at[pl.ds((b0 + i) * N, N)],
                           w_v.at[pl.ds(i * N, N)], s)
          for i, s in ((0, sw0), (1, sw1))]
    hx = [pltpu.async_copy(state_hbm.at[pl.ds((b0 + i) * D * N, D * N)],
                           x_v.at[pl.ds(i * D * N, D * N)], s)
          for i, s in ((0, sx0), (1, sx1))]
    ho = [None, None]
    osem = (so0, so1)

    def zero_body(c, _):
        base = pl.multiple_of(c * 4 * L, L)
        for j in range(4):
            hist_v[pl.ds(base + j * L, L)] = zi
        return 0

    lax.fori_loop(0, (HLEN // L) // 4, zero_body, 0)
    hist_v[pl.ds(N, L)] = zi
    hu.wait()

    for bi in range(2):
        b = b0 + bi
        woff = bi * N
        xoff = bi * D * N
        hw[bi].wait()
        u = plsc.load_gather(u_v, [jnp.full((L,), b, jnp.int32)])

        def scan_body(i, carry):
            a0, a1 = carry
            ib = i * 16
            for di in range(16):
                idx0 = i128 + (woff + ib + di)
                idx1 = i128 + (woff + L * SCH + ib + di)
                a0 = a0 + plsc.load_gather(w_v, [idx0])
                a1 = a1 + plsc.load_gather(w_v, [idx1])
                plsc.store_scatter(w_v, [idx0], a0)
                plsc.store_scatter(w_v, [idx1], a1)
            return (a0, a1)

        lax.fori_loop(0, SCH // 16, scan_body, (zf, zf))

        def hist_body(c, off):
            mbase = c * SCH
            for j in range(SCH // L):
                base = pl.multiple_of(woff + mbase + j * L, L)
                cdf = w_v[pl.ds(base, L)] + off
                mf4 = cdf * float(N)
                mf = mf4 - u
                it = mf.astype(jnp.int32)
                ifl = it - (it.astype(jnp.float32) > mf).astype(jnp.int32)
                c0 = ifl + 1
                c1 = c0 - ((ifl.astype(jnp.float32) + u) > mf4).astype(jnp.int32)
                c2 = c1 + ((c1.astype(jnp.float32) + u) <= mf4).astype(jnp.int32)
                cc = jnp.minimum(c2, N + 8)
                plsc.addupdate_scatter(hist_v, [cc], ones_i)
            tot = plsc.load_gather(
                w_v, [jnp.full((L,), woff + mbase + (SCH - 1), jnp.int32)])
            return off + tot

        lax.fori_loop(0, NSC, hist_body, zf)

        hx[bi].wait()

        def res_body(c, accc):
            base0 = c * 16 * L
            for j in range(16):
                base = pl.multiple_of(base0 + j * L, L)
                hv = hist_v[pl.ds(base, L)]
                hist_v[pl.ds(base, L)] = zi
                ics = plsc.cumsum(hv) + accc
                idxv = jnp.minimum(ics, N - 1) + xoff
                for k in range(D):
                    g = plsc.load_gather(x_v, [idxv + k * N])
                    o_v[pl.ds(xoff + k * N + base, L)] = g
                accc = ics[L - 1]
            return accc

        lax.fori_loop(0, NCH // 16, res_body, 0)
        ho[bi] = pltpu.async_copy(o_v.at[pl.ds(xoff, D * N)],
                                  out_hbm.at[pl.ds(b * D * N, D * N)],
                                  osem[bi])
    ho[0].wait()
    ho[1].wait()


_mesh = plsc.VectorSubcoreMesh(core_axis_name="c", subcore_axis_name="s")

_resample = pl.kernel(
    _resample_body,
    out_type=jax.ShapeDtypeStruct((B * D * N,), jnp.float32),
    mesh=_mesh,
    compiler_params=pltpu.CompilerParams(needs_layout_passes=False,
                                         disable_bounds_checks=True),
    scratch_types=[
        pltpu.VMEM((2 * N,), jnp.float32),
        pltpu.VMEM((B,), jnp.float32),
        pltpu.VMEM((2 * D * N,), jnp.float32),
        pltpu.VMEM((HLEN,), jnp.int32),
        pltpu.VMEM((2 * D * N,), jnp.float32),
        pltpu.SemaphoreType.DMA,
        pltpu.SemaphoreType.DMA,
        pltpu.SemaphoreType.DMA,
        pltpu.SemaphoreType.DMA,
        pltpu.SemaphoreType.DMA,
        pltpu.SemaphoreType.DMA,
        pltpu.SemaphoreType.DMA,
    ],
)


def _obs_logw(y, x):
    return -0.5 * jnp.sum((y[None, :, :] - x) ** 2, axis=-1) / 0.25


def _norm(logw):
    lse = jax.scipy.special.logsumexp(logw, axis=0)
    return logw - lse[None, :]


def kernel(observation, init_noise, trans_noise, resample_u, A,
           n_particles, time_extent):
    state = init_noise
    logw = _obs_logw(observation[0], state)
    lw = _norm(logw)
    w = jnp.exp(lw)
    outputs = [jnp.sum(w[..., None] * state, axis=0)]

    for t in range(1, trans_noise.shape[0] + 1):
        w = jnp.exp(lw)
        w_t = w.T.reshape(B * N)
        state_t = state.transpose(1, 2, 0).reshape(B * D * N)
        res_t = _resample(w_t, resample_u[t - 1], state_t)
        resampled = res_t.reshape(B, D, N).transpose(2, 0, 1)
        state = jnp.einsum('nbd,kd->nbk', resampled, A) + 0.3 * trans_noise[t - 1]
        logw = _obs_logw(observation[t], state)
        lw = _norm(logw)
        w2 = jnp.exp(lw)
        outputs.append(jnp.sum(w2[..., None] * state, axis=0))

    return jnp.stack(outputs, axis=0)
